# baseline (speedup 1.0000x reference)
import functools

import jax
import jax.numpy as jnp
from jax import lax
from jax.experimental import pallas as pl
from jax.experimental.pallas import tpu as pltpu
from jax.experimental.pallas import tpu_sc as plsc

VOCAB = 1000000
EMBED_DIM = 64
BATCH = 16384
SEQ = 200

NUM_CORES = 2
NUM_SUBCORES = 16
NW = NUM_CORES * NUM_SUBCORES

B_TOTAL = BATCH * SEQ
PER_W = B_TOTAL // NW

IDX_ROW = 128
ROWS_PER_CHUNK = 8
CHUNK = IDX_ROW * ROWS_PER_CHUNK
CHUNKS_PER_W = PER_W // CHUNK


def _sc_gather(idx2d, weight):
    mesh = plsc.VectorSubcoreMesh(core_axis_name="c", subcore_axis_name="s")

    @functools.partial(
        pl.kernel,
        out_type=jax.ShapeDtypeStruct((B_TOTAL, EMBED_DIM), jnp.float32),
        mesh=mesh,
        scratch_types=[
            pltpu.VMEM((ROWS_PER_CHUNK, IDX_ROW), jnp.int32),
            pltpu.VMEM((CHUNK, EMBED_DIM), jnp.float32),
            pltpu.SemaphoreType.DMA,
        ],
        compiler_params=pltpu.CompilerParams(use_tc_tiling_on_sc=False),
    )
    def k(idx_hbm, table_hbm, out_hbm, idx_v, rows_v, sem):
        wid = lax.axis_index("s") * NUM_CORES + lax.axis_index("c")
        row0 = wid * (PER_W // IDX_ROW)

        def body(i, carry):
            r0 = row0 + i * ROWS_PER_CHUNK
            pltpu.sync_copy(idx_hbm.at[pl.ds(r0, ROWS_PER_CHUNK)], idx_v)
            cps = [
                pltpu.async_copy(
                    table_hbm.at[idx_v.at[j]],
                    rows_v.at[pl.ds(j * IDX_ROW, IDX_ROW)],
                    sem,
                )
                for j in range(ROWS_PER_CHUNK)
            ]
            for cp in cps:
                cp.wait()
            start = wid * PER_W + i * CHUNK
            pltpu.sync_copy(rows_v, out_hbm.at[pl.ds(start, CHUNK)])
            return carry

        lax.fori_loop(0, CHUNKS_PER_W, body, 0)

    return k(idx2d, weight)


def kernel(input, weight):
    idx2d = input.reshape(B_TOTAL // IDX_ROW, IDX_ROW).astype(jnp.int32)
    out = _sc_gather(idx2d, weight)
    return out.reshape(BATCH, SEQ, EMBED_DIM)


# speedup vs baseline: 1.0092x; 1.0092x over previous
import functools

import jax
import jax.numpy as jnp
from jax import lax
from jax.experimental import pallas as pl
from jax.experimental.pallas import tpu as pltpu
from jax.experimental.pallas import tpu_sc as plsc

VOCAB = 1000000
EMBED_DIM = 64
BATCH = 16384
SEQ = 200

NUM_CORES = 2
NUM_SUBCORES = 16
NW = NUM_CORES * NUM_SUBCORES

B_TOTAL = BATCH * SEQ
PER_W = B_TOTAL // NW

IDX_ROW = 128
ROWS_PER_CHUNK = 4
CHUNK = IDX_ROW * ROWS_PER_CHUNK
NCHUNK = PER_W // CHUNK
NPAIR = NCHUNK // 2


def _sc_gather(idx2d, weight):
    mesh = plsc.VectorSubcoreMesh(core_axis_name="c", subcore_axis_name="s")

    @functools.partial(
        pl.kernel,
        out_type=jax.ShapeDtypeStruct((B_TOTAL, EMBED_DIM), jnp.float32),
        mesh=mesh,
        scratch_types=[
            pltpu.VMEM((ROWS_PER_CHUNK, IDX_ROW), jnp.int32),
            pltpu.VMEM((ROWS_PER_CHUNK, IDX_ROW), jnp.int32),
            pltpu.VMEM((CHUNK, EMBED_DIM), jnp.float32),
            pltpu.VMEM((CHUNK, EMBED_DIM), jnp.float32),
            pltpu.SemaphoreType.DMA,
            pltpu.SemaphoreType.DMA,
            pltpu.SemaphoreType.DMA,
            pltpu.SemaphoreType.DMA,
        ],
        compiler_params=pltpu.CompilerParams(use_tc_tiling_on_sc=False),
    )
    def k(idx_hbm, table_hbm, out_hbm, idx_v0, idx_v1, rows_v0, rows_v1,
          gsem0, gsem1, ssem0, ssem1):
        wid = lax.axis_index("s") * NUM_CORES + lax.axis_index("c")
        row0 = wid * (PER_W // IDX_ROW)
        base = wid * PER_W

        def fire_gather(c, idx_v, rows_v, gsem):
            pltpu.sync_copy(idx_hbm.at[pl.ds(row0 + c * ROWS_PER_CHUNK,
                                             ROWS_PER_CHUNK)], idx_v)
            for j in range(ROWS_PER_CHUNK):
                pltpu.async_copy(
                    table_hbm.at[idx_v.at[j]],
                    rows_v.at[pl.ds(j * IDX_ROW, IDX_ROW)],
                    gsem,
                )

        def wait_gather(idx_v, rows_v, gsem):
            for j in range(ROWS_PER_CHUNK):
                pltpu.make_async_copy(
                    table_hbm.at[idx_v.at[j]],
                    rows_v.at[pl.ds(j * IDX_ROW, IDX_ROW)],
                    gsem,
                ).wait()

        def out_slice(c):
            return out_hbm.at[pl.ds(base + c * CHUNK, CHUNK)]

        fire_gather(0, idx_v0, rows_v0, gsem0)
        fire_gather(1, idx_v1, rows_v1, gsem1)

        def half(i, c, idx_v, rows_v, gsem, ssem):
            wait_gather(idx_v, rows_v, gsem)
            pltpu.async_copy(rows_v, out_slice(c), ssem)

            @pl.when(i < NPAIR - 1)
            def _():
                pltpu.make_async_copy(rows_v, out_slice(c), ssem).wait()
                fire_gather(c + 2, idx_v, rows_v, gsem)

        def body(i, carry):
            half(i, 2 * i, idx_v0, rows_v0, gsem0, ssem0)
            half(i, 2 * i + 1, idx_v1, rows_v1, gsem1, ssem1)
            return carry

        lax.fori_loop(0, NPAIR, body, 0)

        pltpu.make_async_copy(rows_v0, out_slice(NCHUNK - 2), ssem0).wait()
        pltpu.make_async_copy(rows_v1, out_slice(NCHUNK - 1), ssem1).wait()

    return k(idx2d, weight)


def kernel(input, weight):
    idx2d = input.reshape(B_TOTAL // IDX_ROW, IDX_ROW).astype(jnp.int32)
    out = _sc_gather(idx2d, weight)
    return out.reshape(BATCH, SEQ, EMBED_DIM)
